# Initial kernel scaffold; baseline (speedup 1.0000x reference)
import jax
import jax.numpy as jnp
from jax.experimental import pallas as pl


def kernel(x, w_gate, w_noise, expert_base_w, expert_poly_w, base_w1, poly_w1):
    raise NotImplementedError("write your pallas kernel here")


# speedup vs baseline: 3.0866x; 3.0866x over previous
import functools

import jax
import jax.numpy as jnp
from jax.experimental import pallas as pl
from jax.experimental.pallas import tpu as pltpu

C = 96
HW = 224
NPIX = HW * HW
NEXP = 8
CF = 5 * C
EPS_IN = 1e-5
_SQRT2 = 1.4142135623730951


def _features(xv):
    s = xv * (1.0 / (1.0 + jnp.exp(-xv)))
    t = jnp.tanh(xv)
    p2 = 1.5 * t * t - 0.5
    p3 = t * (2.5 * t * t - 1.5)
    ones = jnp.ones_like(xv)
    return s, ones, t, p2, p3


def _gating_kernel(xf_ref, wg_ref, wn_ref, nz_ref, idx_ref, gv_ref, loss_ref):
    pooled = jnp.sum(xf_ref[...], axis=0, keepdims=True) * (1.0 / NPIX)
    clean = jnp.dot(pooled, wg_ref[...], preferred_element_type=jnp.float32)
    raw = jnp.dot(pooled, wn_ref[...], preferred_element_type=jnp.float32)
    std = jnp.logaddexp(raw, 0.0) + 1e-2
    noisy = clean + nz_ref[...] * std

    iota = jax.lax.broadcasted_iota(jnp.int32, (1, NEXP), 1)
    neg = jnp.float32(-1e30)
    m1 = jnp.max(noisy)
    i1 = jnp.min(jnp.where(noisy == m1, iota, NEXP))
    v2 = jnp.where(iota == i1, neg, noisy)
    m2 = jnp.max(v2)
    i2 = jnp.min(jnp.where(v2 == m2, iota, NEXP))
    v3 = jnp.where(iota == i2, neg, v2)
    m3 = jnp.max(v3)

    sel = (iota == i1) | (iota == i2)
    ez = jnp.where(sel, jnp.exp(noisy - m1), 0.0)
    gates = ez * (1.0 / jnp.sum(ez))

    z_in = (clean - m3) / (std * _SQRT2)
    z_out = (clean - m2) / (std * _SQRT2)
    prob_in = 0.5 * (1.0 + jax.lax.erf(z_in))
    prob_out = 0.5 * (1.0 + jax.lax.erf(z_out))
    load = jnp.where(noisy > m3, prob_in, prob_out)

    def cv2(v):
        mu = jnp.sum(v) * (1.0 / NEXP)
        var = jnp.sum((v - mu) ** 2) * (1.0 / (NEXP - 1))
        return var / (mu * mu + 1e-10)

    loss_ref[0] = 0.01 * (cv2(gates) + cv2(load))
    idx_ref[0] = i1
    idx_ref[1] = i2
    gv_ref[0] = jnp.sum(jnp.where(iota == i1, gates, 0.0))
    gv_ref[1] = jnp.sum(jnp.where(iota == i2, gates, 0.0))


def _run_gating(xf, w_gate, w_noise, noise):
    return pl.pallas_call(
        _gating_kernel,
        out_shape=(
            jax.ShapeDtypeStruct((2,), jnp.int32),
            jax.ShapeDtypeStruct((2,), jnp.float32),
            jax.ShapeDtypeStruct((1,), jnp.float32),
        ),
        in_specs=[
            pl.BlockSpec(memory_space=pltpu.MemorySpace.VMEM),
            pl.BlockSpec(memory_space=pltpu.MemorySpace.VMEM),
            pl.BlockSpec(memory_space=pltpu.MemorySpace.VMEM),
            pl.BlockSpec(memory_space=pltpu.MemorySpace.VMEM),
        ],
        out_specs=(
            pl.BlockSpec(memory_space=pltpu.MemorySpace.SMEM),
            pl.BlockSpec(memory_space=pltpu.MemorySpace.SMEM),
            pl.BlockSpec(memory_space=pltpu.MemorySpace.SMEM),
        ),
    )(xf, w_gate, w_noise, noise)


TR = 8
NT = HW // TR


def _expert_conv_kernel(idx_ref, xp_ref, w_ref, y_ref, st_ref, xbuf, sem):
    i = pl.program_id(1)
    cp = pltpu.make_async_copy(xp_ref.at[pl.ds(i * TR, TR + 2)], xbuf, sem)
    cp.start()
    cp.wait()
    xv = xbuf[...]

    s, ones, t, p2, p3 = _features(xv)
    r = jax.lax.broadcasted_iota(jnp.int32, (TR + 2, HW + 2, 1), 0) + i * TR
    c = jax.lax.broadcasted_iota(jnp.int32, (TR + 2, HW + 2, 1), 1)
    mask = ((r >= 1) & (r <= HW) & (c >= 1) & (c <= HW)).astype(jnp.float32)
    f = jnp.concatenate([s, ones * mask, t, p2 * mask, p3], axis=2)
    fw = jnp.concatenate(
        [f[:, 0:HW, :], f[:, 1:HW + 1, :], f[:, 2:HW + 2, :]], axis=2)

    acc = jnp.zeros((TR * HW, C), jnp.float32)
    for dy in range(3):
        a = fw[dy:dy + TR].reshape(TR * HW, 3 * CF)
        acc = acc + jax.lax.dot_general(
            a, w_ref[0, dy], (((1,), (0,)), ((), ())),
            preferred_element_type=jnp.float32)

    y_ref[0] = acc.reshape(TR, HW, C)
    s1 = jnp.sum(acc, axis=0, keepdims=True)
    s2 = jnp.sum(acc * acc, axis=0, keepdims=True)
    upd = jnp.concatenate([s1, s2, jnp.zeros((6, C), jnp.float32)], axis=0)

    @pl.when(i == 0)
    def _():
        st_ref[0] = upd

    @pl.when(i != 0)
    def _():
        st_ref[0] = st_ref[0] + upd


def _run_expert_conv(idx, xp, wm):
    grid_spec = pltpu.PrefetchScalarGridSpec(
        num_scalar_prefetch=1,
        grid=(2, NT),
        in_specs=[
            pl.BlockSpec(memory_space=pl.ANY),
            pl.BlockSpec((1, 3, 3 * CF, C), lambda k, i, idx: (idx[k], 0, 0, 0)),
        ],
        out_specs=(
            pl.BlockSpec((1, TR, HW, C), lambda k, i, idx: (k, i, 0, 0)),
            pl.BlockSpec((1, 8, C), lambda k, i, idx: (k, 0, 0)),
        ),
        scratch_shapes=[
            pltpu.VMEM((TR + 2, HW + 2, C), jnp.float32),
            pltpu.SemaphoreType.DMA,
        ],
    )
    return pl.pallas_call(
        _expert_conv_kernel,
        grid_spec=grid_spec,
        out_shape=(
            jax.ShapeDtypeStruct((2, HW, HW, C), jnp.float32),
            jax.ShapeDtypeStruct((2, 8, C), jnp.float32),
        ),
    )(idx, xp, wm)


TR2 = 16
NT2 = HW // TR2


def _combine_kernel(y_ref, st_ref, gv_ref, w1_ref, z_ref, st2_ref):
    i = pl.program_id(0)
    st = st_ref[...]
    inv_n = 1.0 / NPIX
    m0 = (st[0, 0:1, :] * inv_n).reshape(1, 1, C)
    v0 = (st[0, 1:2, :] * inv_n).reshape(1, 1, C) - m0 * m0
    m1 = (st[1, 0:1, :] * inv_n).reshape(1, 1, C)
    v1 = (st[1, 1:2, :] * inv_n).reshape(1, 1, C) - m1 * m1
    sc0 = gv_ref[0] * jax.lax.rsqrt(v0 + EPS_IN)
    sc1 = gv_ref[1] * jax.lax.rsqrt(v1 + EPS_IN)
    moe = (y_ref[0] - m0) * sc0 + (y_ref[1] - m1) * sc1

    s, ones, t, p2, p3 = _features(moe)
    f2 = jnp.concatenate([s, ones, t, p2, p3], axis=2).reshape(TR2 * HW, CF)
    z = jax.lax.dot_general(f2, w1_ref[...], (((1,), (0,)), ((), ())),
                            preferred_element_type=jnp.float32)
    z_ref[...] = z.reshape(TR2, HW, C)

    s1 = jnp.sum(z, axis=0, keepdims=True)
    s2 = jnp.sum(z * z, axis=0, keepdims=True)
    upd = jnp.concatenate([s1, s2, jnp.zeros((6, C), jnp.float32)], axis=0)

    @pl.when(i == 0)
    def _():
        st2_ref[...] = upd

    @pl.when(i != 0)
    def _():
        st2_ref[...] = st2_ref[...] + upd


def _run_combine(y, st, gv, w1m):
    return pl.pallas_call(
        _combine_kernel,
        grid=(NT2,),
        in_specs=[
            pl.BlockSpec((2, TR2, HW, C), lambda i: (0, i, 0, 0)),
            pl.BlockSpec((2, 8, C), lambda i: (0, 0, 0)),
            pl.BlockSpec(memory_space=pltpu.MemorySpace.SMEM),
            pl.BlockSpec((CF, C), lambda i: (0, 0)),
        ],
        out_specs=(
            pl.BlockSpec((TR2, HW, C), lambda i: (i, 0, 0)),
            pl.BlockSpec((8, C), lambda i: (0, 0)),
        ),
        out_shape=(
            jax.ShapeDtypeStruct((HW, HW, C), jnp.float32),
            jax.ShapeDtypeStruct((8, C), jnp.float32),
        ),
    )(y, st, gv, w1m)


TR3 = 56
NT3 = HW // TR3


def _final_kernel(z_ref, st2_ref, xt_ref, o_ref):
    st = st2_ref[...]
    inv_n = 1.0 / NPIX
    m = (st[0:1, :] * inv_n).reshape(1, 1, C)
    v = (st[1:2, :] * inv_n).reshape(1, 1, C) - m * m
    o_ref[...] = (z_ref[...] - m) * jax.lax.rsqrt(v + EPS_IN) + xt_ref[...]


def _run_final(z, st2, xt):
    return pl.pallas_call(
        _final_kernel,
        grid=(NT3,),
        in_specs=[
            pl.BlockSpec((TR3, HW, C), lambda i: (i, 0, 0)),
            pl.BlockSpec((8, C), lambda i: (0, 0)),
            pl.BlockSpec((TR3, HW, C), lambda i: (i, 0, 0)),
        ],
        out_specs=pl.BlockSpec((TR3, HW, C), lambda i: (i, 0, 0)),
        out_shape=jax.ShapeDtypeStruct((HW, HW, C), jnp.float32),
    )(z, st2, xt)


@jax.jit
def kernel(x, w_gate, w_noise, expert_base_w, expert_poly_w, base_w1, poly_w1):
    xt = jnp.transpose(x[0], (1, 2, 0))
    xp = jnp.pad(xt, ((1, 1), (1, 1), (0, 0)))
    xf = xt.reshape(NPIX, C)
    noise = jax.random.normal(jax.random.key(42), (1, NEXP), jnp.float32)

    wcat = jnp.concatenate([expert_base_w, expert_poly_w], axis=2)
    wm = jnp.transpose(wcat, (0, 3, 4, 2, 1)).reshape(NEXP, 3, 3 * CF, C)
    w1m = jnp.concatenate([base_w1, poly_w1], axis=1)[:, :, 0, 0].T

    idx, gv, loss = _run_gating(xf, w_gate, w_noise, noise)
    y, st = _run_expert_conv(idx, xp, wm)
    z, st2 = _run_combine(y, st, gv, w1m)
    ot = _run_final(z, st2, xt)

    out = jnp.transpose(ot, (2, 0, 1))[None]
    return out, loss[0]


# speedup vs baseline: 3.3265x; 1.0777x over previous
import functools

import jax
import jax.numpy as jnp
from jax.experimental import pallas as pl
from jax.experimental.pallas import tpu as pltpu

C = 96
HW = 224
NPIX = HW * HW
NEXP = 8
CF = 5 * C
EPS_IN = 1e-5
_SQRT2 = 1.4142135623730951


def _features(xv):
    s = xv * (1.0 / (1.0 + jnp.exp(-xv)))
    t = jnp.tanh(xv)
    p2 = 1.5 * t * t - 0.5
    p3 = t * (2.5 * t * t - 1.5)
    ones = jnp.ones_like(xv)
    return s, ones, t, p2, p3


def _gating_kernel(xf_ref, wg_ref, wn_ref, nz_ref, idx_ref, gv_ref, loss_ref):
    pooled = jnp.sum(xf_ref[...], axis=0, keepdims=True) * (1.0 / NPIX)
    clean = jnp.dot(pooled, wg_ref[...], preferred_element_type=jnp.float32)
    raw = jnp.dot(pooled, wn_ref[...], preferred_element_type=jnp.float32)
    std = jnp.logaddexp(raw, 0.0) + 1e-2
    noisy = clean + nz_ref[...] * std

    iota = jax.lax.broadcasted_iota(jnp.int32, (1, NEXP), 1)
    neg = jnp.float32(-1e30)
    m1 = jnp.max(noisy)
    i1 = jnp.min(jnp.where(noisy == m1, iota, NEXP))
    v2 = jnp.where(iota == i1, neg, noisy)
    m2 = jnp.max(v2)
    i2 = jnp.min(jnp.where(v2 == m2, iota, NEXP))
    v3 = jnp.where(iota == i2, neg, v2)
    m3 = jnp.max(v3)

    sel = (iota == i1) | (iota == i2)
    ez = jnp.where(sel, jnp.exp(noisy - m1), 0.0)
    gates = ez * (1.0 / jnp.sum(ez))

    z_in = (clean - m3) / (std * _SQRT2)
    z_out = (clean - m2) / (std * _SQRT2)
    prob_in = 0.5 * (1.0 + jax.lax.erf(z_in))
    prob_out = 0.5 * (1.0 + jax.lax.erf(z_out))
    load = jnp.where(noisy > m3, prob_in, prob_out)

    def cv2(v):
        mu = jnp.sum(v) * (1.0 / NEXP)
        var = jnp.sum((v - mu) ** 2) * (1.0 / (NEXP - 1))
        return var / (mu * mu + 1e-10)

    loss_ref[0] = 0.01 * (cv2(gates) + cv2(load))
    idx_ref[0] = i1
    idx_ref[1] = i2
    gv_ref[0] = jnp.sum(jnp.where(iota == i1, gates, 0.0))
    gv_ref[1] = jnp.sum(jnp.where(iota == i2, gates, 0.0))


def _run_gating(xf, w_gate, w_noise, noise):
    return pl.pallas_call(
        _gating_kernel,
        out_shape=(
            jax.ShapeDtypeStruct((2,), jnp.int32),
            jax.ShapeDtypeStruct((2,), jnp.float32),
            jax.ShapeDtypeStruct((1,), jnp.float32),
        ),
        in_specs=[
            pl.BlockSpec(memory_space=pltpu.MemorySpace.VMEM),
            pl.BlockSpec(memory_space=pltpu.MemorySpace.VMEM),
            pl.BlockSpec(memory_space=pltpu.MemorySpace.VMEM),
            pl.BlockSpec(memory_space=pltpu.MemorySpace.VMEM),
        ],
        out_specs=(
            pl.BlockSpec(memory_space=pltpu.MemorySpace.SMEM),
            pl.BlockSpec(memory_space=pltpu.MemorySpace.SMEM),
            pl.BlockSpec(memory_space=pltpu.MemorySpace.SMEM),
        ),
    )(xf, w_gate, w_noise, noise)


TR = 8
NT = HW // TR


def _expert_conv_kernel(idx_ref, xp_ref, w_ref, y_ref, st_ref, xbuf, sem):
    i = pl.program_id(1)
    cp = pltpu.make_async_copy(xp_ref.at[pl.ds(i * TR, TR + 2)], xbuf, sem)
    cp.start()
    cp.wait()
    xv = xbuf[...]

    s, ones, t, p2, p3 = _features(xv)
    r = jax.lax.broadcasted_iota(jnp.int32, (TR + 2, HW + 2, 1), 0) + i * TR
    c = jax.lax.broadcasted_iota(jnp.int32, (TR + 2, HW + 2, 1), 1)
    mask = ((r >= 1) & (r <= HW) & (c >= 1) & (c <= HW)).astype(jnp.float32)
    f = jnp.concatenate(
        [s, ones * mask, t, p2 * mask, p3], axis=2).astype(jnp.bfloat16)
    fw = jnp.concatenate(
        [f[:, 0:HW, :], f[:, 1:HW + 1, :], f[:, 2:HW + 2, :]], axis=2)

    acc = jnp.zeros((TR * HW, C), jnp.float32)
    for dy in range(3):
        a = fw[dy:dy + TR].reshape(TR * HW, 3 * CF)
        acc = acc + jax.lax.dot_general(
            a, w_ref[0, dy], (((1,), (0,)), ((), ())),
            preferred_element_type=jnp.float32)

    y_ref[0] = acc.reshape(TR, HW, C).astype(jnp.bfloat16)
    s1 = jnp.sum(acc, axis=0, keepdims=True)
    s2 = jnp.sum(acc * acc, axis=0, keepdims=True)
    upd = jnp.concatenate([s1, s2, jnp.zeros((6, C), jnp.float32)], axis=0)

    @pl.when(i == 0)
    def _():
        st_ref[0] = upd

    @pl.when(i != 0)
    def _():
        st_ref[0] = st_ref[0] + upd


def _run_expert_conv(idx, xp, wm):
    grid_spec = pltpu.PrefetchScalarGridSpec(
        num_scalar_prefetch=1,
        grid=(2, NT),
        in_specs=[
            pl.BlockSpec(memory_space=pl.ANY),
            pl.BlockSpec((1, 3, 3 * CF, C), lambda k, i, idx: (idx[k], 0, 0, 0)),
        ],
        out_specs=(
            pl.BlockSpec((1, TR, HW, C), lambda k, i, idx: (k, i, 0, 0)),
            pl.BlockSpec((1, 8, C), lambda k, i, idx: (k, 0, 0)),
        ),
        scratch_shapes=[
            pltpu.VMEM((TR + 2, HW + 2, C), jnp.float32),
            pltpu.SemaphoreType.DMA,
        ],
    )
    return pl.pallas_call(
        _expert_conv_kernel,
        grid_spec=grid_spec,
        out_shape=(
            jax.ShapeDtypeStruct((2, HW, HW, C), jnp.bfloat16),
            jax.ShapeDtypeStruct((2, 8, C), jnp.float32),
        ),
    )(idx, xp, wm)


TR2 = 16
NT2 = HW // TR2


def _combine_kernel(y_ref, st_ref, gv_ref, w1_ref, z_ref, st2_ref):
    i = pl.program_id(0)
    st = st_ref[...]
    inv_n = 1.0 / NPIX
    m0 = (st[0, 0:1, :] * inv_n).reshape(1, 1, C)
    v0 = (st[0, 1:2, :] * inv_n).reshape(1, 1, C) - m0 * m0
    m1 = (st[1, 0:1, :] * inv_n).reshape(1, 1, C)
    v1 = (st[1, 1:2, :] * inv_n).reshape(1, 1, C) - m1 * m1
    sc0 = gv_ref[0] * jax.lax.rsqrt(v0 + EPS_IN)
    sc1 = gv_ref[1] * jax.lax.rsqrt(v1 + EPS_IN)
    y0 = y_ref[0].astype(jnp.float32)
    y1 = y_ref[1].astype(jnp.float32)
    moe = (y0 - m0) * sc0 + (y1 - m1) * sc1

    s, ones, t, p2, p3 = _features(moe)
    f2 = jnp.concatenate(
        [s, ones, t, p2, p3], axis=2).astype(jnp.bfloat16).reshape(TR2 * HW, CF)
    z = jax.lax.dot_general(f2, w1_ref[...], (((1,), (0,)), ((), ())),
                            preferred_element_type=jnp.float32)
    z_ref[...] = z.reshape(TR2, HW, C)

    s1 = jnp.sum(z, axis=0, keepdims=True)
    s2 = jnp.sum(z * z, axis=0, keepdims=True)
    upd = jnp.concatenate([s1, s2, jnp.zeros((6, C), jnp.float32)], axis=0)

    @pl.when(i == 0)
    def _():
        st2_ref[...] = upd

    @pl.when(i != 0)
    def _():
        st2_ref[...] = st2_ref[...] + upd


def _run_combine(y, st, gv, w1m):
    return pl.pallas_call(
        _combine_kernel,
        grid=(NT2,),
        in_specs=[
            pl.BlockSpec((2, TR2, HW, C), lambda i: (0, i, 0, 0)),
            pl.BlockSpec((2, 8, C), lambda i: (0, 0, 0)),
            pl.BlockSpec(memory_space=pltpu.MemorySpace.SMEM),
            pl.BlockSpec((CF, C), lambda i: (0, 0)),
        ],
        out_specs=(
            pl.BlockSpec((TR2, HW, C), lambda i: (i, 0, 0)),
            pl.BlockSpec((8, C), lambda i: (0, 0)),
        ),
        out_shape=(
            jax.ShapeDtypeStruct((HW, HW, C), jnp.float32),
            jax.ShapeDtypeStruct((8, C), jnp.float32),
        ),
    )(y, st, gv, w1m)


TR3 = 56
NT3 = HW // TR3


def _final_kernel(z_ref, st2_ref, xt_ref, o_ref):
    st = st2_ref[...]
    inv_n = 1.0 / NPIX
    m = (st[0:1, :] * inv_n).reshape(1, 1, C)
    v = (st[1:2, :] * inv_n).reshape(1, 1, C) - m * m
    o_ref[...] = (z_ref[...] - m) * jax.lax.rsqrt(v + EPS_IN) + xt_ref[...]


def _run_final(z, st2, xt):
    return pl.pallas_call(
        _final_kernel,
        grid=(NT3,),
        in_specs=[
            pl.BlockSpec((TR3, HW, C), lambda i: (i, 0, 0)),
            pl.BlockSpec((8, C), lambda i: (0, 0)),
            pl.BlockSpec((TR3, HW, C), lambda i: (i, 0, 0)),
        ],
        out_specs=pl.BlockSpec((TR3, HW, C), lambda i: (i, 0, 0)),
        out_shape=jax.ShapeDtypeStruct((HW, HW, C), jnp.float32),
    )(z, st2, xt)


@jax.jit
def kernel(x, w_gate, w_noise, expert_base_w, expert_poly_w, base_w1, poly_w1):
    xt = jnp.transpose(x[0], (1, 2, 0))
    xp = jnp.pad(xt, ((1, 1), (1, 1), (0, 0)))
    xf = xt.reshape(NPIX, C)
    noise = jax.random.normal(jax.random.key(42), (1, NEXP), jnp.float32)

    wcat = jnp.concatenate([expert_base_w, expert_poly_w], axis=2)
    wm = jnp.transpose(wcat, (0, 3, 4, 2, 1)).reshape(
        NEXP, 3, 3 * CF, C).astype(jnp.bfloat16)
    w1m = jnp.concatenate(
        [base_w1, poly_w1], axis=1)[:, :, 0, 0].T.astype(jnp.bfloat16)

    idx, gv, loss = _run_gating(xf, w_gate, w_noise, noise)
    y, st = _run_expert_conv(idx, xp, wm)
    z, st2 = _run_combine(y, st, gv, w1m)
    ot = _run_final(z, st2, xt)

    out = jnp.transpose(ot, (2, 0, 1))[None]
    return out, loss[0]


# speedup vs baseline: 6.2224x; 1.8705x over previous
import jax
import jax.numpy as jnp
from jax.experimental import pallas as pl
from jax.experimental.pallas import tpu as pltpu

C = 96
HW = 224
NPIX = HW * HW
NEXP = 8
CP = 4 * C
EPS_IN = 1e-5
_SQRT2 = 1.4142135623730951


NB1 = 8
BLK1 = NPIX // NB1


def _gating_math(pooled, wg_ref, wn_ref, nz_ref, idx_ref, gv_ref, loss_ref):
    clean = jnp.dot(pooled, wg_ref[...], preferred_element_type=jnp.float32)
    raw = jnp.dot(pooled, wn_ref[...], preferred_element_type=jnp.float32)
    std = jnp.logaddexp(raw, 0.0) + 1e-2
    noisy = clean + nz_ref[...] * std

    iota = jax.lax.broadcasted_iota(jnp.int32, (1, NEXP), 1)
    neg = jnp.float32(-1e30)
    m1 = jnp.max(noisy)
    i1 = jnp.min(jnp.where(noisy == m1, iota, NEXP))
    v2 = jnp.where(iota == i1, neg, noisy)
    m2 = jnp.max(v2)
    i2 = jnp.min(jnp.where(v2 == m2, iota, NEXP))
    v3 = jnp.where(iota == i2, neg, v2)
    m3 = jnp.max(v3)

    sel = (iota == i1) | (iota == i2)
    ez = jnp.where(sel, jnp.exp(noisy - m1), 0.0)
    gates = ez * (1.0 / jnp.sum(ez))

    z_in = (clean - m3) / (std * _SQRT2)
    z_out = (clean - m2) / (std * _SQRT2)
    prob_in = 0.5 * (1.0 + jax.lax.erf(z_in))
    prob_out = 0.5 * (1.0 + jax.lax.erf(z_out))
    load = jnp.where(noisy > m3, prob_in, prob_out)

    def cv2(v):
        mu = jnp.sum(v) * (1.0 / NEXP)
        var = jnp.sum((v - mu) ** 2) * (1.0 / (NEXP - 1))
        return var / (mu * mu + 1e-10)

    loss_ref[0] = 0.01 * (cv2(gates) + cv2(load))
    idx_ref[0] = i1
    idx_ref[1] = i2
    gv_ref[0] = jnp.sum(jnp.where(iota == i1, gates, 0.0))
    gv_ref[1] = jnp.sum(jnp.where(iota == i2, gates, 0.0))


def _gating_kernel(x2_ref, wg_ref, wn_ref, nz_ref, idx_ref, gv_ref, loss_ref,
                   ps_ref):
    i = pl.program_id(0)
    p = jnp.sum(x2_ref[...], axis=1, keepdims=True)

    @pl.when(i == 0)
    def _():
        ps_ref[...] = p

    @pl.when(i != 0)
    def _():
        ps_ref[...] = ps_ref[...] + p

    @pl.when(i == NB1 - 1)
    def _():
        pooled = jnp.transpose(ps_ref[...]) * (1.0 / NPIX)
        _gating_math(pooled, wg_ref, wn_ref, nz_ref, idx_ref, gv_ref, loss_ref)


def _run_gating(x2, w_gate, w_noise, noise):
    return pl.pallas_call(
        _gating_kernel,
        grid=(NB1,),
        in_specs=[
            pl.BlockSpec((C, BLK1), lambda i: (0, i)),
            pl.BlockSpec((C, NEXP), lambda i: (0, 0)),
            pl.BlockSpec((C, NEXP), lambda i: (0, 0)),
            pl.BlockSpec((1, NEXP), lambda i: (0, 0)),
        ],
        out_specs=(
            pl.BlockSpec(memory_space=pltpu.MemorySpace.SMEM),
            pl.BlockSpec(memory_space=pltpu.MemorySpace.SMEM),
            pl.BlockSpec(memory_space=pltpu.MemorySpace.SMEM),
        ),
        out_shape=(
            jax.ShapeDtypeStruct((2,), jnp.int32),
            jax.ShapeDtypeStruct((2,), jnp.float32),
            jax.ShapeDtypeStruct((1,), jnp.float32),
        ),
        scratch_shapes=[pltpu.VMEM((C, 1), jnp.float32)],
    )(x2, w_gate, w_noise, noise)


TR = 16
NT = HW // TR
C2 = 2 * C
KB = 3 * C
KP = 3 * CP


def _expert_conv_kernel(idx_ref, xm_ref, xt_ref, xb_ref,
                        wb0_ref, wb1_ref, wp0_ref, wp1_ref,
                        y_ref, st_ref, wb, wp):
    i = pl.program_id(0)

    @pl.when(i == 0)
    def _():
        wb[:, :, 0:C] = wb0_ref[0]
        wb[:, :, C:C2] = wb1_ref[0]
        wp[:, :, 0:C] = wp0_ref[0]
        wp[:, :, C:C2] = wp1_ref[0]

    mt = jnp.transpose(xm_ref[...].reshape(C, TR * HW)).reshape(TR, HW, C)
    tt = jnp.transpose(xt_ref[...][:, 7, :]).reshape(1, HW, C)
    bt = jnp.transpose(xb_ref[...][:, 0, :]).reshape(1, HW, C)
    xv = jnp.concatenate([tt, mt, bt], axis=0)

    s = xv * (1.0 / (1.0 + jnp.exp(-xv)))
    t = jnp.tanh(xv)
    p2 = 1.5 * t * t - 0.5
    p3 = t * (2.5 * t * t - 1.5)
    r = jax.lax.broadcasted_iota(jnp.int32, (TR + 2, HW, 1), 0) + (i * TR - 1)
    rmask = (r >= 0) & (r < HW)
    s = jnp.where(rmask, s, 0.0)
    t = jnp.where(rmask, t, 0.0)
    p2 = jnp.where(rmask, p2, 0.0)
    p3 = jnp.where(rmask, p3, 0.0)
    ones = jnp.where(rmask, 1.0, 0.0) * jnp.ones_like(xv)

    sb = s.astype(jnp.bfloat16)
    fp = jnp.concatenate([ones, t, p2, p3], axis=2).astype(jnp.bfloat16)
    zc1 = jnp.zeros((TR + 2, 1, C), jnp.bfloat16)
    zc4 = jnp.zeros((TR + 2, 1, CP), jnp.bfloat16)
    sw = jnp.concatenate([zc1, sb, zc1], axis=1)
    pw = jnp.concatenate([zc4, fp, zc4], axis=1)
    fwb = jnp.concatenate([sw[:, 0:HW], sw[:, 1:HW + 1], sw[:, 2:HW + 2]],
                          axis=2)
    fwp = jnp.concatenate([pw[:, 0:HW], pw[:, 1:HW + 1], pw[:, 2:HW + 2]],
                          axis=2)

    acc = jnp.zeros((TR * HW, C2), jnp.float32)
    for dy in range(3):
        ab = fwb[dy:dy + TR].reshape(TR * HW, KB)
        ap = fwp[dy:dy + TR].reshape(TR * HW, KP)
        acc = acc + jax.lax.dot_general(
            ab, wb[dy], (((1,), (0,)), ((), ())),
            preferred_element_type=jnp.float32)
        acc = acc + jax.lax.dot_general(
            ap, wp[dy], (((1,), (0,)), ((), ())),
            preferred_element_type=jnp.float32)

    y_ref[...] = acc.reshape(TR, HW, C2).astype(jnp.bfloat16)
    s1 = jnp.sum(acc, axis=0, keepdims=True)
    s2 = jnp.sum(acc * acc, axis=0, keepdims=True)
    upd = jnp.concatenate([s1, s2, jnp.zeros((6, C2), jnp.float32)], axis=0)

    @pl.when(i == 0)
    def _():
        st_ref[...] = upd

    @pl.when(i != 0)
    def _():
        st_ref[...] = st_ref[...] + upd


def _run_expert_conv(idx, x3, wmb, wmp):
    grid_spec = pltpu.PrefetchScalarGridSpec(
        num_scalar_prefetch=1,
        grid=(NT,),
        in_specs=[
            pl.BlockSpec((C, TR, HW), lambda i, idx: (0, i, 0)),
            pl.BlockSpec((C, 8, HW),
                         lambda i, idx: (0, jnp.maximum(i * (TR // 8) - 1, 0), 0)),
            pl.BlockSpec((C, 8, HW),
                         lambda i, idx: (0, jnp.minimum((i + 1) * (TR // 8), HW // 8 - 1), 0)),
            pl.BlockSpec((1, 3, KB, C), lambda i, idx: (idx[0], 0, 0, 0)),
            pl.BlockSpec((1, 3, KB, C), lambda i, idx: (idx[1], 0, 0, 0)),
            pl.BlockSpec((1, 3, KP, C), lambda i, idx: (idx[0], 0, 0, 0)),
            pl.BlockSpec((1, 3, KP, C), lambda i, idx: (idx[1], 0, 0, 0)),
        ],
        out_specs=(
            pl.BlockSpec((TR, HW, C2), lambda i, idx: (i, 0, 0)),
            pl.BlockSpec((8, C2), lambda i, idx: (0, 0)),
        ),
        scratch_shapes=[
            pltpu.VMEM((3, KB, C2), jnp.bfloat16),
            pltpu.VMEM((3, KP, C2), jnp.bfloat16),
        ],
    )
    return pl.pallas_call(
        _expert_conv_kernel,
        grid_spec=grid_spec,
        out_shape=(
            jax.ShapeDtypeStruct((HW, HW, C2), jnp.bfloat16),
            jax.ShapeDtypeStruct((8, C2), jnp.float32),
        ),
    )(idx, x3, x3, x3, wmb, wmb, wmp, wmp)


TR2 = 16
NT2 = HW // TR2
CF = 5 * C


def _combine_kernel(y_ref, st_ref, gv_ref, w1_ref, z_ref, st2_ref):
    i = pl.program_id(0)
    st = st_ref[...]
    inv_n = 1.0 / NPIX
    m = (st[0:1, :] * inv_n).reshape(1, 1, C2)
    v = (st[1:2, :] * inv_n).reshape(1, 1, C2) - m * m
    lane = jax.lax.broadcasted_iota(jnp.int32, (1, 1, C2), 2)
    g = jnp.where(lane < C, gv_ref[0], gv_ref[1])
    sc = g * jax.lax.rsqrt(v + EPS_IN)
    off = -m * sc
    yn = y_ref[...].astype(jnp.float32) * sc + off
    moe = yn[:, :, 0:C] + yn[:, :, C:C2]

    s = moe * (1.0 / (1.0 + jnp.exp(-moe)))
    t = jnp.tanh(moe)
    p2 = 1.5 * t * t - 0.5
    p3 = t * (2.5 * t * t - 1.5)
    ones = jnp.ones_like(moe)
    f2 = jnp.concatenate(
        [s, ones, t, p2, p3], axis=2).astype(jnp.bfloat16).reshape(TR2 * HW, CF)
    z = jax.lax.dot_general(f2, w1_ref[...], (((1,), (0,)), ((), ())),
                            preferred_element_type=jnp.float32)
    z_ref[...] = z.reshape(TR2, HW, C)

    s1 = jnp.sum(z, axis=0, keepdims=True)
    s2 = jnp.sum(z * z, axis=0, keepdims=True)
    upd = jnp.concatenate([s1, s2, jnp.zeros((6, C), jnp.float32)], axis=0)

    @pl.when(i == 0)
    def _():
        st2_ref[...] = upd

    @pl.when(i != 0)
    def _():
        st2_ref[...] = st2_ref[...] + upd


def _run_combine(y, st, gv, w1m):
    return pl.pallas_call(
        _combine_kernel,
        grid=(NT2,),
        in_specs=[
            pl.BlockSpec((TR2, HW, C2), lambda i: (i, 0, 0)),
            pl.BlockSpec((8, C2), lambda i: (0, 0)),
            pl.BlockSpec(memory_space=pltpu.MemorySpace.SMEM),
            pl.BlockSpec((CF, C), lambda i: (0, 0)),
        ],
        out_specs=(
            pl.BlockSpec((TR2, HW, C), lambda i: (i, 0, 0)),
            pl.BlockSpec((8, C), lambda i: (0, 0)),
        ),
        out_shape=(
            jax.ShapeDtypeStruct((HW, HW, C), jnp.float32),
            jax.ShapeDtypeStruct((8, C), jnp.float32),
        ),
    )(y, st, gv, w1m)


TR3 = 28
NT3 = HW // TR3


def _final_kernel(z_ref, st2_ref, x2_ref, o_ref):
    st = st2_ref[...]
    inv_n = 1.0 / NPIX
    m = (st[0:1, :] * inv_n).reshape(1, 1, C)
    v = (st[1:2, :] * inv_n).reshape(1, 1, C) - m * m
    zn = (z_ref[...] - m) * jax.lax.rsqrt(v + EPS_IN)
    o_ref[...] = jnp.transpose(zn.reshape(TR3 * HW, C)) + x2_ref[...]


def _run_final(z, st2, x2):
    return pl.pallas_call(
        _final_kernel,
        grid=(NT3,),
        in_specs=[
            pl.BlockSpec((TR3, HW, C), lambda i: (i, 0, 0)),
            pl.BlockSpec((8, C), lambda i: (0, 0)),
            pl.BlockSpec((C, TR3 * HW), lambda i: (0, i)),
        ],
        out_specs=pl.BlockSpec((C, TR3 * HW), lambda i: (0, i)),
        out_shape=jax.ShapeDtypeStruct((C, NPIX), jnp.float32),
    )(z, st2, x2)


@jax.jit
def kernel(x, w_gate, w_noise, expert_base_w, expert_poly_w, base_w1, poly_w1):
    x2 = x.reshape(C, NPIX)
    noise = jax.random.normal(jax.random.key(42), (1, NEXP), jnp.float32)

    wmb = jnp.transpose(expert_base_w, (0, 3, 4, 2, 1)).reshape(
        NEXP, 3, KB, C).astype(jnp.bfloat16)
    wmp = jnp.transpose(expert_poly_w, (0, 3, 4, 2, 1)).reshape(
        NEXP, 3, KP, C).astype(jnp.bfloat16)
    w1m = jnp.concatenate(
        [base_w1, poly_w1], axis=1)[:, :, 0, 0].T.astype(jnp.bfloat16)

    idx, gv, loss = _run_gating(x2, w_gate, w_noise, noise)
    y, st = _run_expert_conv(idx, x.reshape(C, HW, HW), wmb, wmp)
    z, st2 = _run_combine(y, st, gv, w1m)
    o2 = _run_final(z, st2, x2)

    return o2.reshape(1, C, HW, HW), loss[0]


# speedup vs baseline: 6.3677x; 1.0233x over previous
import jax
import jax.numpy as jnp
from jax.experimental import pallas as pl
from jax.experimental.pallas import tpu as pltpu

C = 96
HW = 224
NPIX = HW * HW
NEXP = 8
CP = 4 * C
EPS_IN = 1e-5
_SQRT2 = 1.4142135623730951


NB1 = 8
BLK1 = NPIX // NB1


def _gating_math(pooled, wg_ref, wn_ref, nz_ref, idx_ref, gv_ref, loss_ref):
    clean = jnp.dot(pooled, wg_ref[...], preferred_element_type=jnp.float32)
    raw = jnp.dot(pooled, wn_ref[...], preferred_element_type=jnp.float32)
    std = jnp.logaddexp(raw, 0.0) + 1e-2
    noisy = clean + nz_ref[...] * std

    iota = jax.lax.broadcasted_iota(jnp.int32, (1, NEXP), 1)
    neg = jnp.float32(-1e30)
    m1 = jnp.max(noisy)
    i1 = jnp.min(jnp.where(noisy == m1, iota, NEXP))
    v2 = jnp.where(iota == i1, neg, noisy)
    m2 = jnp.max(v2)
    i2 = jnp.min(jnp.where(v2 == m2, iota, NEXP))
    v3 = jnp.where(iota == i2, neg, v2)
    m3 = jnp.max(v3)

    sel = (iota == i1) | (iota == i2)
    ez = jnp.where(sel, jnp.exp(noisy - m1), 0.0)
    gates = ez * (1.0 / jnp.sum(ez))

    z_in = (clean - m3) / (std * _SQRT2)
    z_out = (clean - m2) / (std * _SQRT2)
    prob_in = 0.5 * (1.0 + jax.lax.erf(z_in))
    prob_out = 0.5 * (1.0 + jax.lax.erf(z_out))
    load = jnp.where(noisy > m3, prob_in, prob_out)

    def cv2(v):
        mu = jnp.sum(v) * (1.0 / NEXP)
        var = jnp.sum((v - mu) ** 2) * (1.0 / (NEXP - 1))
        return var / (mu * mu + 1e-10)

    loss_ref[0] = 0.01 * (cv2(gates) + cv2(load))
    idx_ref[0] = i1
    idx_ref[1] = i2
    gv_ref[0] = jnp.sum(jnp.where(iota == i1, gates, 0.0))
    gv_ref[1] = jnp.sum(jnp.where(iota == i2, gates, 0.0))


def _gating_kernel(x2_ref, wg_ref, wn_ref, nz_ref, idx_ref, gv_ref, loss_ref,
                   ps_ref):
    i = pl.program_id(0)
    p = jnp.sum(x2_ref[...], axis=1, keepdims=True)

    @pl.when(i == 0)
    def _():
        ps_ref[...] = p

    @pl.when(i != 0)
    def _():
        ps_ref[...] = ps_ref[...] + p

    @pl.when(i == NB1 - 1)
    def _():
        pooled = jnp.transpose(ps_ref[...]) * (1.0 / NPIX)
        _gating_math(pooled, wg_ref, wn_ref, nz_ref, idx_ref, gv_ref, loss_ref)


def _run_gating(x2, w_gate, w_noise, noise):
    return pl.pallas_call(
        _gating_kernel,
        grid=(NB1,),
        in_specs=[
            pl.BlockSpec((C, BLK1), lambda i: (0, i)),
            pl.BlockSpec((C, NEXP), lambda i: (0, 0)),
            pl.BlockSpec((C, NEXP), lambda i: (0, 0)),
            pl.BlockSpec((1, NEXP), lambda i: (0, 0)),
        ],
        out_specs=(
            pl.BlockSpec(memory_space=pltpu.MemorySpace.SMEM),
            pl.BlockSpec(memory_space=pltpu.MemorySpace.SMEM),
            pl.BlockSpec(memory_space=pltpu.MemorySpace.SMEM),
        ),
        out_shape=(
            jax.ShapeDtypeStruct((2,), jnp.int32),
            jax.ShapeDtypeStruct((2,), jnp.float32),
            jax.ShapeDtypeStruct((1,), jnp.float32),
        ),
        scratch_shapes=[pltpu.VMEM((C, 1), jnp.float32)],
    )(x2, w_gate, w_noise, noise)


TR = 16
NT = HW // TR
C2 = 2 * C
KB = 3 * C
KP = 3 * CP


def _expert_conv_kernel(xm_ref, xt_ref, xb_ref,
                        wb0_ref, wb1_ref, wp0_ref, wp1_ref,
                        y_ref, st_ref, wb, wp):
    i = pl.program_id(0)

    @pl.when(i == 0)
    def _():
        wb[:, :, 0:C] = wb0_ref[0]
        wb[:, :, C:C2] = wb1_ref[0]
        wp[:, :, 0:C] = wp0_ref[0]
        wp[:, :, C:C2] = wp1_ref[0]

    mt = jnp.transpose(xm_ref[...].reshape(C, TR * HW)).reshape(TR, HW, C)
    tt = jnp.transpose(xt_ref[...][:, 7, :]).reshape(1, HW, C)
    bt = jnp.transpose(xb_ref[...][:, 0, :]).reshape(1, HW, C)
    xv = jnp.concatenate([tt, mt, bt], axis=0)

    s = xv * (1.0 / (1.0 + jnp.exp(-xv)))
    t = jnp.tanh(xv)
    p2 = 1.5 * t * t - 0.5
    p3 = t * (2.5 * t * t - 1.5)
    r = jax.lax.broadcasted_iota(jnp.int32, (TR + 2, HW, 1), 0) + (i * TR - 1)
    rmask = (r >= 0) & (r < HW)
    s = jnp.where(rmask, s, 0.0)
    t = jnp.where(rmask, t, 0.0)
    p2 = jnp.where(rmask, p2, 0.0)
    p3 = jnp.where(rmask, p3, 0.0)
    ones = jnp.where(rmask, 1.0, 0.0) * jnp.ones_like(xv)

    sb = s.astype(jnp.bfloat16)
    fp = jnp.concatenate([ones, t, p2, p3], axis=2).astype(jnp.bfloat16)
    zc1 = jnp.zeros((TR + 2, 1, C), jnp.bfloat16)
    zc4 = jnp.zeros((TR + 2, 1, CP), jnp.bfloat16)
    sw = jnp.concatenate([zc1, sb, zc1], axis=1)
    pw = jnp.concatenate([zc4, fp, zc4], axis=1)
    fwb = jnp.concatenate([sw[:, 0:HW], sw[:, 1:HW + 1], sw[:, 2:HW + 2]],
                          axis=2)
    fwp = jnp.concatenate([pw[:, 0:HW], pw[:, 1:HW + 1], pw[:, 2:HW + 2]],
                          axis=2)

    acc = jnp.zeros((TR * HW, C2), jnp.float32)
    for dy in range(3):
        ab = fwb[dy:dy + TR].reshape(TR * HW, KB)
        ap = fwp[dy:dy + TR].reshape(TR * HW, KP)
        acc = acc + jax.lax.dot_general(
            ab, wb[dy], (((1,), (0,)), ((), ())),
            preferred_element_type=jnp.float32)
        acc = acc + jax.lax.dot_general(
            ap, wp[dy], (((1,), (0,)), ((), ())),
            preferred_element_type=jnp.float32)

    y_ref[...] = acc.reshape(TR, HW, C2).astype(jnp.bfloat16)
    s1 = jnp.sum(acc, axis=0, keepdims=True)
    s2 = jnp.sum(acc * acc, axis=0, keepdims=True)
    upd = jnp.concatenate([s1, s2, jnp.zeros((6, C2), jnp.float32)], axis=0)

    @pl.when(i == 0)
    def _():
        st_ref[...] = upd

    @pl.when(i != 0)
    def _():
        st_ref[...] = st_ref[...] + upd


def _run_expert_conv(x3, wmb, wmp):
    return pl.pallas_call(
        _expert_conv_kernel,
        grid=(NT,),
        in_specs=[
            pl.BlockSpec((C, TR, HW), lambda i: (0, i, 0)),
            pl.BlockSpec((C, 8, HW),
                         lambda i: (0, jnp.maximum(i * (TR // 8) - 1, 0), 0)),
            pl.BlockSpec((C, 8, HW),
                         lambda i: (0, jnp.minimum((i + 1) * (TR // 8), HW // 8 - 1), 0)),
            pl.BlockSpec((1, 3, KB, C), lambda i: (0, 0, 0, 0)),
            pl.BlockSpec((1, 3, KB, C), lambda i: (1, 0, 0, 0)),
            pl.BlockSpec((1, 3, KP, C), lambda i: (0, 0, 0, 0)),
            pl.BlockSpec((1, 3, KP, C), lambda i: (1, 0, 0, 0)),
        ],
        out_specs=(
            pl.BlockSpec((TR, HW, C2), lambda i: (i, 0, 0)),
            pl.BlockSpec((8, C2), lambda i: (0, 0)),
        ),
        scratch_shapes=[
            pltpu.VMEM((3, KB, C2), jnp.bfloat16),
            pltpu.VMEM((3, KP, C2), jnp.bfloat16),
        ],
        out_shape=(
            jax.ShapeDtypeStruct((HW, HW, C2), jnp.bfloat16),
            jax.ShapeDtypeStruct((8, C2), jnp.float32),
        ),
    )(x3, x3, x3, wmb, wmb, wmp, wmp)


TR2 = 16
NT2 = HW // TR2
CF = 5 * C


def _combine_kernel(y_ref, st_ref, gv_ref, w1_ref, z_ref, st2_ref):
    i = pl.program_id(0)
    st = st_ref[...]
    inv_n = 1.0 / NPIX
    m = (st[0:1, :] * inv_n).reshape(1, 1, C2)
    v = (st[1:2, :] * inv_n).reshape(1, 1, C2) - m * m
    lane = jax.lax.broadcasted_iota(jnp.int32, (1, 1, C2), 2)
    g = jnp.where(lane < C, gv_ref[0], gv_ref[1])
    sc = g * jax.lax.rsqrt(v + EPS_IN)
    off = -m * sc
    yn = y_ref[...].astype(jnp.float32) * sc + off
    moe = yn[:, :, 0:C] + yn[:, :, C:C2]

    s = moe * (1.0 / (1.0 + jnp.exp(-moe)))
    t = jnp.tanh(moe)
    p2 = 1.5 * t * t - 0.5
    p3 = t * (2.5 * t * t - 1.5)
    ones = jnp.ones_like(moe)
    f2 = jnp.concatenate(
        [s, ones, t, p2, p3], axis=2).astype(jnp.bfloat16).reshape(TR2 * HW, CF)
    z = jax.lax.dot_general(f2, w1_ref[...], (((1,), (0,)), ((), ())),
                            preferred_element_type=jnp.float32)
    z_ref[...] = z.reshape(TR2, HW, C)

    s1 = jnp.sum(z, axis=0, keepdims=True)
    s2 = jnp.sum(z * z, axis=0, keepdims=True)
    upd = jnp.concatenate([s1, s2, jnp.zeros((6, C), jnp.float32)], axis=0)

    @pl.when(i == 0)
    def _():
        st2_ref[...] = upd

    @pl.when(i != 0)
    def _():
        st2_ref[...] = st2_ref[...] + upd


def _run_combine(y, st, gv, w1m):
    return pl.pallas_call(
        _combine_kernel,
        grid=(NT2,),
        in_specs=[
            pl.BlockSpec((TR2, HW, C2), lambda i: (i, 0, 0)),
            pl.BlockSpec((8, C2), lambda i: (0, 0)),
            pl.BlockSpec(memory_space=pltpu.MemorySpace.SMEM),
            pl.BlockSpec((CF, C), lambda i: (0, 0)),
        ],
        out_specs=(
            pl.BlockSpec((TR2, HW, C), lambda i: (i, 0, 0)),
            pl.BlockSpec((8, C), lambda i: (0, 0)),
        ),
        out_shape=(
            jax.ShapeDtypeStruct((HW, HW, C), jnp.float32),
            jax.ShapeDtypeStruct((8, C), jnp.float32),
        ),
    )(y, st, gv, w1m)


TR3 = 28
NT3 = HW // TR3


def _final_kernel(z_ref, st2_ref, x2_ref, o_ref):
    st = st2_ref[...]
    inv_n = 1.0 / NPIX
    m = (st[0:1, :] * inv_n).reshape(1, 1, C)
    v = (st[1:2, :] * inv_n).reshape(1, 1, C) - m * m
    zn = (z_ref[...] - m) * jax.lax.rsqrt(v + EPS_IN)
    o_ref[...] = jnp.transpose(zn.reshape(TR3 * HW, C)) + x2_ref[...]


def _run_final(z, st2, x2):
    return pl.pallas_call(
        _final_kernel,
        grid=(NT3,),
        in_specs=[
            pl.BlockSpec((TR3, HW, C), lambda i: (i, 0, 0)),
            pl.BlockSpec((8, C), lambda i: (0, 0)),
            pl.BlockSpec((C, TR3 * HW), lambda i: (0, i)),
        ],
        out_specs=pl.BlockSpec((C, TR3 * HW), lambda i: (0, i)),
        out_shape=jax.ShapeDtypeStruct((C, NPIX), jnp.float32),
    )(z, st2, x2)


@jax.jit
def kernel(x, w_gate, w_noise, expert_base_w, expert_poly_w, base_w1, poly_w1):
    x2 = x.reshape(C, NPIX)
    noise = jax.random.normal(jax.random.key(42), (1, NEXP), jnp.float32)
    w1m = jnp.concatenate(
        [base_w1, poly_w1], axis=1)[:, :, 0, 0].T.astype(jnp.bfloat16)

    idx, gv, loss = _run_gating(x2, w_gate, w_noise, noise)

    wmb = jnp.transpose(expert_base_w[idx], (0, 3, 4, 2, 1)).reshape(
        2, 3, KB, C).astype(jnp.bfloat16)
    wmp = jnp.transpose(expert_poly_w[idx], (0, 3, 4, 2, 1)).reshape(
        2, 3, KP, C).astype(jnp.bfloat16)

    y, st = _run_expert_conv(x.reshape(C, HW, HW), wmb, wmp)
    z, st2 = _run_combine(y, st, gv, w1m)
    o2 = _run_final(z, st2, x2)

    return o2.reshape(1, C, HW, HW), loss[0]
